# baseline (speedup 1.0000x reference)
import functools

import jax
import jax.numpy as jnp
from jax import lax
from jax.experimental import pallas as pl
from jax.experimental.pallas import tpu as pltpu
from jax.experimental.pallas import tpu_sc as plsc

N = 10000
B = 8
F = 20000
NC = 1
NS = 16
NW = NC * NS
RPT = 640
NPAD = NS * RPT
FPT = 1280
FPAD = NW * FPT
KCH = 60
CHW = 128
NRING = 4
BIG = 40.0
DIRS = ((0, 1), (1, 2), (2, 0), (1, 0), (2, 1), (0, 2))

NR = NPAD // 8
NVALID = N // 8


def _sc_body(pred_hbm, faces_hbm, out_hbm, pc_hbm,
             predt, vbuf, fbuf, gbuf, ridx, cidx, pc16_s, acc_s,
             sem0, sem1):
    s = lax.axis_index("s")
    r0 = s * RPT
    lane = lax.iota(jnp.int32, 16)
    lane_lo = lane < 8

    for b in range(B):
        pltpu.sync_copy(pred_hbm.at[b, pl.ds(r0, RPT)],
                        predt.at[pl.ds(b * RPT, RPT)])

    def sig_step(i, _):
        v = plsc.load_gather(predt, [(lane & 7) * RPT + i])
        sig = 1.0 / (1.0 + jnp.exp(-v))
        row = jnp.where(lane_lo, sig, 1.0 - sig)
        row = jnp.where(r0 + i < N, row, 0.0)
        vbuf[i] = row
        return 0

    lax.fori_loop(0, RPT, sig_step, 0)
    pltpu.sync_copy(vbuf, pc16_s.at[pl.ds(r0, RPT)])
    pltpu.sync_copy(vbuf, acc_s.at[pl.ds(r0, RPT)])
    pltpu.sync_copy(vbuf, pc_hbm.at[pl.ds(r0, RPT)])

    pltpu.sync_copy(faces_hbm.at[s], fbuf)

    def bld_step(v, _):
        base3 = (v * 16 + lane) * 3
        g = [plsc.load_gather(fbuf, [base3 + k]) for k in range(3)]
        row = v // 8
        co = (v % 8) * 16
        for d, (rc, cc) in enumerate(DIRS):
            ridx[10 * d + row, pl.ds(co, 16)] = g[rc]
            cidx[10 * d + row, pl.ds(co, 16)] = g[cc]
        return 0

    lax.fori_loop(0, FPT // 16, bld_step, 0)
    plsc.subcore_barrier()

    pltpu.async_copy(pc16_s.at[cidx.at[0]], gbuf.at[0], sem0)

    def edge_step(h, _):
        j = h * 2
        pltpu.async_copy(
            pc16_s.at[cidx.at[j + 1]], gbuf.at[(j + 1) % NRING], sem1)
        pltpu.make_async_copy(
            pc16_s.at[cidx.at[j]], gbuf.at[j % NRING], sem0).wait()
        pltpu.sync_copy(gbuf.at[j % NRING], acc_s.at[ridx.at[j]], add=True)

        @pl.when(j + 2 < KCH)
        def _():
            pltpu.async_copy(
                pc16_s.at[cidx.at[j + 2]], gbuf.at[(j + 2) % NRING], sem0)

        pltpu.make_async_copy(
            pc16_s.at[cidx.at[j + 1]], gbuf.at[(j + 1) % NRING], sem1).wait()
        pltpu.sync_copy(
            gbuf.at[(j + 1) % NRING], acc_s.at[ridx.at[j + 1]], add=True)
        return 0

    lax.fori_loop(0, KCH // 2, edge_step, 0)
    plsc.subcore_barrier()

    pltpu.sync_copy(acc_s.at[pl.ds(r0, RPT)], out_hbm.at[pl.ds(r0, RPT)])


def _reduce_body(part_ref, pc_ref, out_ref):
    pc = pc_ref[...]
    t = part_ref[...]
    rowv = lax.broadcasted_iota(jnp.int32, (NR, 128), 0) < NVALID
    tm = jnp.where(rowv, t, 0.0)
    pcm = jnp.where(rowv, pc, 0.0)

    def fold(x, op):
        m = x[:, 0:16]
        for k in range(1, 8):
            m = op(m, x[:, 16 * k:16 * (k + 1)])
        return m

    def widen(x16):
        return jnp.concatenate([x16] * 8, axis=1)

    m16 = fold(jnp.max(tm, axis=0, keepdims=True), jnp.maximum)
    sum16 = fold(jnp.sum(tm, axis=0, keepdims=True), jnp.add)
    m128 = widen(m16)

    iso128 = jnp.sum(jnp.abs(pcm - tm / (m128 + 1e-6)),
                     axis=0, keepdims=True)
    iso16 = fold(iso128, jnp.add)

    isoc = iso16[:, 0:8] + iso16[:, 8:16]
    norm = sum16[:, 0:8] + sum16[:, 8:16] + 0.001
    pen = jnp.log1p(isoc / norm)
    out_ref[...] = (jnp.sum(pen) / B).reshape(1, 1)


@jax.jit
def kernel(pred, faces):
    faces_pad = jnp.concatenate(
        [faces, jnp.full((FPAD - F, 3), N, jnp.int32)]).reshape(NW, FPT * 3)

    mesh = plsc.VectorSubcoreMesh(
        core_axis_name="c", subcore_axis_name="s",
        num_cores=NC, num_subcores=NS)
    part, pc16 = pl.kernel(
        _sc_body,
        out_type=(jax.ShapeDtypeStruct((NPAD, 16), jnp.float32),
                  jax.ShapeDtypeStruct((NPAD, 16), jnp.float32)),
        mesh=mesh,
        scratch_types=[
            pltpu.VMEM((B * RPT,), jnp.float32),
            pltpu.VMEM((RPT, 16), jnp.float32),
            pltpu.VMEM((FPT * 3,), jnp.int32),
            pltpu.VMEM((NRING, CHW, 16), jnp.float32),
            pltpu.VMEM((KCH, CHW), jnp.int32),
            pltpu.VMEM((KCH, CHW), jnp.int32),
            pltpu.VMEM_SHARED((NPAD, 16), jnp.float32),
            pltpu.VMEM_SHARED((NPAD, 16), jnp.float32),
            pltpu.SemaphoreType.DMA,
            pltpu.SemaphoreType.DMA,
        ],
        compiler_params=pltpu.CompilerParams(
            use_tc_tiling_on_sc=False, needs_layout_passes=False),
    )(pred, faces_pad)

    loss = pl.pallas_call(
        _reduce_body,
        out_shape=jax.ShapeDtypeStruct((1, 1), jnp.float32),
    )(part.reshape(NR, 128), pc16.reshape(NR, 128))
    return loss[0, 0]


# speedup vs baseline: 1.0344x; 1.0344x over previous
import functools

import jax
import jax.numpy as jnp
from jax import lax
from jax.experimental import pallas as pl
from jax.experimental.pallas import tpu as pltpu
from jax.experimental.pallas import tpu_sc as plsc

N = 10000
B = 8
F = 20000
NC = 2
NS = 16
NW = NC * NS
RPT = 640
NPAD = NS * RPT
FPT = 640
FPAD = NW * FPT
KCH = 30
RPD = FPT // 128
CHW = 128
NRING = 4
BIG = 40.0
DIRS = ((0, 1), (1, 2), (2, 0), (1, 0), (2, 1), (0, 2))

NR = NPAD // 8
NVALID = N // 8


def _sc_body(pred_hbm, faces_hbm, out_hbm, pc_hbm,
             predt, vbuf, fbuf, gbuf, ridx, cidx, pc16_s, acc_s,
             sem0, sem1):
    c = lax.axis_index("c")
    s = lax.axis_index("s")
    w = c * NS + s
    r0 = s * RPT
    lane = lax.iota(jnp.int32, 16)
    lane_lo = lane < 8

    for b in range(B):
        pltpu.sync_copy(pred_hbm.at[b, pl.ds(r0, RPT)],
                        predt.at[pl.ds(b * RPT, RPT)])

    def sig_step(i, _):
        v = plsc.load_gather(predt, [(lane & 7) * RPT + i])
        sig = 1.0 / (1.0 + jnp.exp(-v))
        row = jnp.where(lane_lo, sig, 1.0 - sig)
        row = jnp.where(r0 + i < N, row, 0.0)
        vbuf[i] = row
        return 0

    lax.fori_loop(0, RPT, sig_step, 0)
    pltpu.sync_copy(vbuf, pc16_s.at[pl.ds(r0, RPT)])

    @pl.when(c == 0)
    def _():
        pltpu.sync_copy(vbuf, acc_s.at[pl.ds(r0, RPT)])
        pltpu.sync_copy(vbuf, pc_hbm.at[pl.ds(r0, RPT)])

    @pl.when(c != 0)
    def _():
        def z_step(i, _):
            vbuf[i] = jnp.zeros((16,), jnp.float32)
            return 0
        lax.fori_loop(0, RPT, z_step, 0)
        pltpu.sync_copy(vbuf, acc_s.at[pl.ds(r0, RPT)])

    pltpu.sync_copy(faces_hbm.at[w], fbuf)

    def bld_step(v, _):
        base3 = (v * 16 + lane) * 3
        g = [plsc.load_gather(fbuf, [base3 + k]) for k in range(3)]
        row = v // 8
        co = (v % 8) * 16
        for d, (rc, cc) in enumerate(DIRS):
            ridx[RPD * d + row, pl.ds(co, 16)] = g[rc]
            cidx[RPD * d + row, pl.ds(co, 16)] = g[cc]
        return 0

    lax.fori_loop(0, FPT // 16, bld_step, 0)
    plsc.subcore_barrier()

    pltpu.async_copy(pc16_s.at[cidx.at[0]], gbuf.at[0], sem0)

    def edge_step(h, _):
        j = h * 2
        pltpu.async_copy(
            pc16_s.at[cidx.at[j + 1]], gbuf.at[(j + 1) % NRING], sem1)
        pltpu.make_async_copy(
            pc16_s.at[cidx.at[j]], gbuf.at[j % NRING], sem0).wait()
        pltpu.sync_copy(gbuf.at[j % NRING], acc_s.at[ridx.at[j]], add=True)

        @pl.when(j + 2 < KCH)
        def _():
            pltpu.async_copy(
                pc16_s.at[cidx.at[j + 2]], gbuf.at[(j + 2) % NRING], sem0)

        pltpu.make_async_copy(
            pc16_s.at[cidx.at[j + 1]], gbuf.at[(j + 1) % NRING], sem1).wait()
        pltpu.sync_copy(
            gbuf.at[(j + 1) % NRING], acc_s.at[ridx.at[j + 1]], add=True)
        return 0

    lax.fori_loop(0, KCH // 2, edge_step, 0)
    plsc.subcore_barrier()

    pltpu.sync_copy(acc_s.at[pl.ds(r0, RPT)], out_hbm.at[c, pl.ds(r0, RPT)])


def _reduce_body(part_ref, pc_ref, out_ref):
    pc = pc_ref[...]
    t = part_ref[0:NR] + part_ref[NR:2 * NR]
    rowv = lax.broadcasted_iota(jnp.int32, (NR, 128), 0) < NVALID
    tm = jnp.where(rowv, t, 0.0)
    pcm = jnp.where(rowv, pc, 0.0)

    def fold(x, op):
        m = x[:, 0:16]
        for k in range(1, 8):
            m = op(m, x[:, 16 * k:16 * (k + 1)])
        return m

    def widen(x16):
        return jnp.concatenate([x16] * 8, axis=1)

    m16 = fold(jnp.max(tm, axis=0, keepdims=True), jnp.maximum)
    sum16 = fold(jnp.sum(tm, axis=0, keepdims=True), jnp.add)
    m128 = widen(m16)

    iso128 = jnp.sum(jnp.abs(pcm - tm / (m128 + 1e-6)),
                     axis=0, keepdims=True)
    iso16 = fold(iso128, jnp.add)

    isoc = iso16[:, 0:8] + iso16[:, 8:16]
    norm = sum16[:, 0:8] + sum16[:, 8:16] + 0.001
    pen = jnp.log1p(isoc / norm)
    out_ref[...] = (jnp.sum(pen) / B).reshape(1, 1)


@jax.jit
def kernel(pred, faces):
    faces_pad = jnp.concatenate(
        [faces, jnp.full((FPAD - F, 3), N, jnp.int32)]).reshape(NW, FPT * 3)

    mesh = plsc.VectorSubcoreMesh(
        core_axis_name="c", subcore_axis_name="s",
        num_cores=NC, num_subcores=NS)
    part, pc16 = pl.kernel(
        _sc_body,
        out_type=(jax.ShapeDtypeStruct((NC, NPAD, 16), jnp.float32),
                  jax.ShapeDtypeStruct((NPAD, 16), jnp.float32)),
        mesh=mesh,
        scratch_types=[
            pltpu.VMEM((B * RPT,), jnp.float32),
            pltpu.VMEM((RPT, 16), jnp.float32),
            pltpu.VMEM((FPT * 3,), jnp.int32),
            pltpu.VMEM((NRING, CHW, 16), jnp.float32),
            pltpu.VMEM((KCH, CHW), jnp.int32),
            pltpu.VMEM((KCH, CHW), jnp.int32),
            pltpu.VMEM_SHARED((NPAD, 16), jnp.float32),
            pltpu.VMEM_SHARED((NPAD, 16), jnp.float32),
            pltpu.SemaphoreType.DMA,
            pltpu.SemaphoreType.DMA,
        ],
        compiler_params=pltpu.CompilerParams(
            use_tc_tiling_on_sc=False, needs_layout_passes=False),
    )(pred, faces_pad)

    loss = pl.pallas_call(
        _reduce_body,
        out_shape=jax.ShapeDtypeStruct((1, 1), jnp.float32),
    )(part.reshape(NC * NR, 128), pc16.reshape(NR, 128))
    return loss[0, 0]


# speedup vs baseline: 1.8133x; 1.7531x over previous
import functools

import jax
import jax.numpy as jnp
from jax import lax
from jax.experimental import pallas as pl
from jax.experimental.pallas import tpu as pltpu
from jax.experimental.pallas import tpu_sc as plsc

N = 10000
B = 8
F = 20000
NC = 2
NS = 16
NW = NC * NS
RPT = 640
NPAD = NS * RPT
FPT = 640
FPAD = NW * FPT
KCH = 30
RPD = FPT // 128
CHW = 128
NRING = 4
DIRS = ((0, 1), (1, 2), (2, 0), (1, 0), (2, 1), (0, 2))

NR = NPAD // 8
NVALID = N // 8


def _sig_body(pred_ref, out_ref):
    x = pred_ref[...]
    s = 1.0 / (1.0 + jnp.exp(-x))
    big = jnp.concatenate([s, 1.0 - s], axis=0)
    valid = lax.broadcasted_iota(jnp.int32, (2 * B, NPAD), 1) < N
    big = jnp.where(valid, big, 0.0)
    out_ref[...] = big.T


def _sc_body(pc_hbm, ridx_hbm, cidx_hbm, out_hbm,
             gbuf, ridx, cidx, pc16_s, acc_s, sem0, sem1):
    c = lax.axis_index("c")
    s = lax.axis_index("s")
    w = c * NS + s
    r0 = s * RPT

    pltpu.sync_copy(pc_hbm.at[pl.ds(r0, RPT)], pc16_s.at[pl.ds(r0, RPT)])
    pltpu.sync_copy(pc_hbm.at[pl.ds(r0, RPT)], acc_s.at[pl.ds(r0, RPT)])
    pltpu.sync_copy(ridx_hbm.at[w], ridx)
    pltpu.sync_copy(cidx_hbm.at[w], cidx)
    plsc.subcore_barrier()

    pltpu.async_copy(pc16_s.at[cidx.at[0]], gbuf.at[0], sem0)

    def edge_step(h, _):
        j = h * 2
        pltpu.async_copy(
            pc16_s.at[cidx.at[j + 1]], gbuf.at[(j + 1) % NRING], sem1)
        pltpu.make_async_copy(
            pc16_s.at[cidx.at[j]], gbuf.at[j % NRING], sem0).wait()
        pltpu.sync_copy(gbuf.at[j % NRING], acc_s.at[ridx.at[j]], add=True)

        @pl.when(j + 2 < KCH)
        def _():
            pltpu.async_copy(
                pc16_s.at[cidx.at[j + 2]], gbuf.at[(j + 2) % NRING], sem0)

        pltpu.make_async_copy(
            pc16_s.at[cidx.at[j + 1]], gbuf.at[(j + 1) % NRING], sem1).wait()
        pltpu.sync_copy(
            gbuf.at[(j + 1) % NRING], acc_s.at[ridx.at[j + 1]], add=True)
        return 0

    lax.fori_loop(0, KCH // 2, edge_step, 0)
    plsc.subcore_barrier()

    pltpu.sync_copy(acc_s.at[pl.ds(r0, RPT)], out_hbm.at[c, pl.ds(r0, RPT)])


def _reduce_body(part_ref, pc_ref, out_ref):
    pc = pc_ref[...]
    t = part_ref[0:NR] + part_ref[NR:2 * NR] - pc
    rowv = lax.broadcasted_iota(jnp.int32, (NR, 128), 0) < NVALID
    tm = jnp.where(rowv, t, 0.0)
    pcm = jnp.where(rowv, pc, 0.0)

    def fold(x, op):
        m = x[:, 0:16]
        for k in range(1, 8):
            m = op(m, x[:, 16 * k:16 * (k + 1)])
        return m

    def widen(x16):
        return jnp.concatenate([x16] * 8, axis=1)

    m16 = fold(jnp.max(tm, axis=0, keepdims=True), jnp.maximum)
    sum16 = fold(jnp.sum(tm, axis=0, keepdims=True), jnp.add)
    m128 = widen(m16)

    iso128 = jnp.sum(jnp.abs(pcm - tm / (m128 + 1e-6)),
                     axis=0, keepdims=True)
    iso16 = fold(iso128, jnp.add)

    isoc = iso16[:, 0:8] + iso16[:, 8:16]
    norm = sum16[:, 0:8] + sum16[:, 8:16] + 0.001
    pen = jnp.log1p(isoc / norm)
    out_ref[...] = (jnp.sum(pen) / B).reshape(1, 1)


@jax.jit
def kernel(pred, faces):
    pred_pad = jnp.pad(pred, ((0, 0), (0, NPAD - N)))
    faces_pad = jnp.concatenate(
        [faces, jnp.full((FPAD - F, 3), N, jnp.int32)]).reshape(NW, FPT, 3)
    ridx = jnp.stack(
        [faces_pad[:, :, rc].reshape(NW, RPD, CHW) for rc, _ in DIRS],
        axis=1).reshape(NW, KCH, CHW)
    cidx = jnp.stack(
        [faces_pad[:, :, cc].reshape(NW, RPD, CHW) for _, cc in DIRS],
        axis=1).reshape(NW, KCH, CHW)

    pc16 = pl.pallas_call(
        _sig_body,
        out_shape=jax.ShapeDtypeStruct((NPAD, 16), jnp.float32),
    )(pred_pad)

    mesh = plsc.VectorSubcoreMesh(
        core_axis_name="c", subcore_axis_name="s",
        num_cores=NC, num_subcores=NS)
    part = pl.kernel(
        _sc_body,
        out_type=jax.ShapeDtypeStruct((NC, NPAD, 16), jnp.float32),
        mesh=mesh,
        scratch_types=[
            pltpu.VMEM((NRING, CHW, 16), jnp.float32),
            pltpu.VMEM((KCH, CHW), jnp.int32),
            pltpu.VMEM((KCH, CHW), jnp.int32),
            pltpu.VMEM_SHARED((NPAD, 16), jnp.float32),
            pltpu.VMEM_SHARED((NPAD, 16), jnp.float32),
            pltpu.SemaphoreType.DMA,
            pltpu.SemaphoreType.DMA,
        ],
        compiler_params=pltpu.CompilerParams(
            use_tc_tiling_on_sc=False, needs_layout_passes=False),
    )(pc16, ridx, cidx)

    loss = pl.pallas_call(
        _reduce_body,
        out_shape=jax.ShapeDtypeStruct((1, 1), jnp.float32),
    )(part.reshape(NC * NR, 128), pc16.reshape(NR, 128))
    return loss[0, 0]


# speedup vs baseline: 1.9867x; 1.0956x over previous
import functools

import jax
import jax.numpy as jnp
from jax import lax
from jax.experimental import pallas as pl
from jax.experimental.pallas import tpu as pltpu
from jax.experimental.pallas import tpu_sc as plsc

N = 10000
B = 8
F = 20000
NC = 2
NS = 16
NW = NC * NS
RPT = 640
NPAD = NS * RPT
FPT = 640
FPAD = NW * FPT
KCH = 30
NIDX = 15
RPD = FPT // 128
CHW = 128
NRING = 4
RB = (0, 1, 2, 1, 2, 0)
CB = (1, 2, 0, 0, 1, 2)
SPERM = tuple(5 * RB[j // RPD] + j % RPD for j in range(KCH))
GPERM = tuple(5 * CB[j // RPD] + j % RPD for j in range(KCH))

NR = NPAD // 8
NVALID = N // 8


def _sig_body(xin_ref, out_ref):
    x = xin_ref[...]
    sub = lax.broadcasted_iota(jnp.int32, (128, NR), 0)
    hi = (sub % 16) >= 8
    val = 1.0 / (1.0 + jnp.exp(jnp.where(hi, x, -x)))
    v = (sub // 16) + 8 * lax.broadcasted_iota(jnp.int32, (128, NR), 1)
    val = jnp.where(v < N, val, 0.0)
    out_ref[...] = val.T


def _sc_body(pc_hbm, idx_hbm, out_hbm,
             gbuf, idx, pc16_s, acc_s, sem0, sem1):
    c = lax.axis_index("c")
    s = lax.axis_index("s")
    w = c * NS + s
    r0 = s * RPT

    pltpu.sync_copy(pc_hbm.at[pl.ds(r0, RPT)], pc16_s.at[pl.ds(r0, RPT)])
    pltpu.sync_copy(pc_hbm.at[pl.ds(r0, RPT)], acc_s.at[pl.ds(r0, RPT)])
    pltpu.sync_copy(idx_hbm.at[w], idx)
    plsc.subcore_barrier()

    pltpu.async_copy(pc16_s.at[idx.at[GPERM[0]]], gbuf.at[0], sem0)
    for j in range(0, KCH, 2):
        pltpu.async_copy(
            pc16_s.at[idx.at[GPERM[j + 1]]], gbuf.at[(j + 1) % NRING], sem1)
        pltpu.make_async_copy(
            pc16_s.at[idx.at[GPERM[j]]], gbuf.at[j % NRING], sem0).wait()
        pltpu.sync_copy(gbuf.at[j % NRING],
                        acc_s.at[idx.at[SPERM[j]]], add=True)
        if j + 2 < KCH:
            pltpu.async_copy(
                pc16_s.at[idx.at[GPERM[j + 2]]], gbuf.at[(j + 2) % NRING],
                sem0)
        pltpu.make_async_copy(
            pc16_s.at[idx.at[GPERM[j + 1]]], gbuf.at[(j + 1) % NRING],
            sem1).wait()
        pltpu.sync_copy(gbuf.at[(j + 1) % NRING],
                        acc_s.at[idx.at[SPERM[j + 1]]], add=True)
    plsc.subcore_barrier()

    pltpu.sync_copy(acc_s.at[pl.ds(r0, RPT)], out_hbm.at[c, pl.ds(r0, RPT)])


def _reduce_body(part_ref, pc_ref, out_ref):
    pc = pc_ref[...]
    t = part_ref[0:NR] + part_ref[NR:2 * NR] - pc
    rowv = lax.broadcasted_iota(jnp.int32, (NR, 128), 0) < NVALID
    tm = jnp.where(rowv, t, 0.0)
    pcm = jnp.where(rowv, pc, 0.0)

    def fold(x, op):
        m = x[:, 0:16]
        for k in range(1, 8):
            m = op(m, x[:, 16 * k:16 * (k + 1)])
        return m

    def widen(x16):
        return jnp.concatenate([x16] * 8, axis=1)

    m16 = fold(jnp.max(tm, axis=0, keepdims=True), jnp.maximum)
    sum16 = fold(jnp.sum(tm, axis=0, keepdims=True), jnp.add)
    m128 = widen(m16)

    iso128 = jnp.sum(jnp.abs(pcm - tm / (m128 + 1e-6)),
                     axis=0, keepdims=True)
    iso16 = fold(iso128, jnp.add)

    isoc = iso16[:, 0:8] + iso16[:, 8:16]
    norm = sum16[:, 0:8] + sum16[:, 8:16] + 0.001
    pen = jnp.log1p(isoc / norm)
    out_ref[...] = (jnp.sum(pen) / B).reshape(1, 1)


@jax.jit
def kernel(pred, faces):
    pred_pad = jnp.pad(pred, ((0, 0), (0, NPAD - N)))
    x3 = pred_pad.reshape(B, NR, 8).transpose(2, 0, 1)
    xin = jnp.concatenate([x3, x3], axis=1).reshape(128, NR)
    faces_pad = jnp.concatenate(
        [faces, jnp.full((FPAD - F, 3), N, jnp.int32)]).reshape(NW, FPT, 3)
    idx = jnp.stack(
        [faces_pad[:, :, k].reshape(NW, RPD, CHW) for k in range(3)],
        axis=1).reshape(NW, NIDX, CHW)

    pc128 = pl.pallas_call(
        _sig_body,
        out_shape=jax.ShapeDtypeStruct((NR, 128), jnp.float32),
    )(xin)

    mesh = plsc.VectorSubcoreMesh(
        core_axis_name="c", subcore_axis_name="s",
        num_cores=NC, num_subcores=NS)
    part = pl.kernel(
        _sc_body,
        out_type=jax.ShapeDtypeStruct((NC, NPAD, 16), jnp.float32),
        mesh=mesh,
        scratch_types=[
            pltpu.VMEM((NRING, CHW, 16), jnp.float32),
            pltpu.VMEM((NIDX, CHW), jnp.int32),
            pltpu.VMEM_SHARED((NPAD, 16), jnp.float32),
            pltpu.VMEM_SHARED((NPAD, 16), jnp.float32),
            pltpu.SemaphoreType.DMA,
            pltpu.SemaphoreType.DMA,
        ],
        compiler_params=pltpu.CompilerParams(
            use_tc_tiling_on_sc=False, needs_layout_passes=False),
    )(pc128.reshape(NPAD, 16), idx)

    loss = pl.pallas_call(
        _reduce_body,
        out_shape=jax.ShapeDtypeStruct((1, 1), jnp.float32),
    )(part.reshape(NC * NR, 128), pc128)
    return loss[0, 0]


# speedup vs baseline: 2.2582x; 1.1367x over previous
import functools

import jax
import jax.numpy as jnp
from jax import lax
from jax.experimental import pallas as pl
from jax.experimental.pallas import tpu as pltpu
from jax.experimental.pallas import tpu_sc as plsc

N = 10000
B = 8
F = 20000
NC = 2
NS = 16
NW = NC * NS
RPT = 640
NPAD = NS * RPT
FPT = 640
FPAD = NW * FPT
KCH = 30
NIDX = 15
RPD = FPT // 128
CHW = 128
NRING = 4
RB = (0, 1, 2, 1, 2, 0)
CB = (1, 2, 0, 0, 1, 2)
SPERM = tuple(5 * RB[j // RPD] + j % RPD for j in range(KCH))
GPERM = tuple(5 * CB[j // RPD] + j % RPD for j in range(KCH))

NR = NPAD // 8


def _sig_body(pred_ref, out_ref):
    x = pred_ref[...]
    riota = lax.broadcasted_iota(jnp.int32, (B, NR), 1)
    parts = []
    for m in range(8):
        xm = x[:, m * NR:(m + 1) * NR]
        sm = 1.0 / (1.0 + jnp.exp(-xm))
        v = riota < (N - NR * m)
        parts.append(jnp.where(v, sm, 0.0))
        parts.append(jnp.where(v, 1.0 - sm, 0.0))
    val = jnp.concatenate(parts, axis=0)
    out_ref[...] = val.T


def _sc_body(pc_hbm, idx_hbm, out_hbm,
             gbuf, idx, pc16_s, acc_s, sem0, sem1):
    c = lax.axis_index("c")
    s = lax.axis_index("s")
    w = c * NS + s
    r0 = s * RPT

    pltpu.sync_copy(pc_hbm.at[pl.ds(r0, RPT)], pc16_s.at[pl.ds(r0, RPT)])
    pltpu.sync_copy(pc_hbm.at[pl.ds(r0, RPT)], acc_s.at[pl.ds(r0, RPT)])
    pltpu.sync_copy(idx_hbm.at[w], idx)
    plsc.subcore_barrier()

    pltpu.async_copy(pc16_s.at[idx.at[GPERM[0]]], gbuf.at[0], sem0)
    for j in range(0, KCH, 2):
        pltpu.async_copy(
            pc16_s.at[idx.at[GPERM[j + 1]]], gbuf.at[(j + 1) % NRING], sem1)
        pltpu.make_async_copy(
            pc16_s.at[idx.at[GPERM[j]]], gbuf.at[j % NRING], sem0).wait()
        pltpu.sync_copy(gbuf.at[j % NRING],
                        acc_s.at[idx.at[SPERM[j]]], add=True)
        if j + 2 < KCH:
            pltpu.async_copy(
                pc16_s.at[idx.at[GPERM[j + 2]]], gbuf.at[(j + 2) % NRING],
                sem0)
        pltpu.make_async_copy(
            pc16_s.at[idx.at[GPERM[j + 1]]], gbuf.at[(j + 1) % NRING],
            sem1).wait()
        pltpu.sync_copy(gbuf.at[(j + 1) % NRING],
                        acc_s.at[idx.at[SPERM[j + 1]]], add=True)
    plsc.subcore_barrier()

    pltpu.sync_copy(acc_s.at[pl.ds(r0, RPT)], out_hbm.at[c, pl.ds(r0, RPT)])


def _reduce_body(part_ref, pc_ref, out_ref):
    pc = pc_ref[...]
    t = part_ref[0:NR] + part_ref[NR:2 * NR] - pc
    lane = lax.broadcasted_iota(jnp.int32, (NR, 128), 1)
    q = lax.broadcasted_iota(jnp.int32, (NR, 128), 0)
    rowv = q < (N - NR * (lane // 16))
    tm = jnp.where(rowv, t, 0.0)
    pcm = jnp.where(rowv, pc, 0.0)

    def fold(x, op):
        m = x[:, 0:16]
        for k in range(1, 8):
            m = op(m, x[:, 16 * k:16 * (k + 1)])
        return m

    def widen(x16):
        return jnp.concatenate([x16] * 8, axis=1)

    m16 = fold(jnp.max(tm, axis=0, keepdims=True), jnp.maximum)
    sum16 = fold(jnp.sum(tm, axis=0, keepdims=True), jnp.add)
    m128 = widen(m16)

    iso128 = jnp.sum(jnp.abs(pcm - tm / (m128 + 1e-6)),
                     axis=0, keepdims=True)
    iso16 = fold(iso128, jnp.add)

    isoc = iso16[:, 0:8] + iso16[:, 8:16]
    norm = sum16[:, 0:8] + sum16[:, 8:16] + 0.001
    pen = jnp.log1p(isoc / norm)
    out_ref[...] = (jnp.sum(pen) / B).reshape(1, 1)


@jax.jit
def kernel(pred, faces):
    pred_pad = jnp.pad(pred, ((0, 0), (0, NPAD - N)))
    faces_pad = jnp.concatenate(
        [faces, jnp.full((FPAD - F, 3), N, jnp.int32)]).reshape(NW, FPT, 3)
    ids = jnp.stack(
        [faces_pad[:, :, k].reshape(NW, RPD, CHW) for k in range(3)],
        axis=1).reshape(NW, NIDX, CHW)
    idx = 8 * (ids % NR) + ids // NR

    pc128 = pl.pallas_call(
        _sig_body,
        out_shape=jax.ShapeDtypeStruct((NR, 128), jnp.float32),
    )(pred_pad)

    mesh = plsc.VectorSubcoreMesh(
        core_axis_name="c", subcore_axis_name="s",
        num_cores=NC, num_subcores=NS)
    part = pl.kernel(
        _sc_body,
        out_type=jax.ShapeDtypeStruct((NC, NPAD, 16), jnp.float32),
        mesh=mesh,
        scratch_types=[
            pltpu.VMEM((NRING, CHW, 16), jnp.float32),
            pltpu.VMEM((NIDX, CHW), jnp.int32),
            pltpu.VMEM_SHARED((NPAD, 16), jnp.float32),
            pltpu.VMEM_SHARED((NPAD, 16), jnp.float32),
            pltpu.SemaphoreType.DMA,
            pltpu.SemaphoreType.DMA,
        ],
        compiler_params=pltpu.CompilerParams(
            use_tc_tiling_on_sc=False, needs_layout_passes=False),
    )(pc128.reshape(NPAD, 16), idx)

    loss = pl.pallas_call(
        _reduce_body,
        out_shape=jax.ShapeDtypeStruct((1, 1), jnp.float32),
    )(part.reshape(NC * NR, 128), pc128)
    return loss[0, 0]
